# baseline (speedup 1.0000x reference)
import jax
import jax.numpy as jnp
from jax import lax
from jax.experimental import pallas as pl
from jax.experimental.pallas import tpu as pltpu
from jax.experimental.pallas import tpu_sc as plsc

N_CORES = 2
N_SUB = 16
CH = 128
SUPE = 1024


def _front_body(f_ref, wp_ref, w1_ref, b1_ref, o_ref):
    x = lax.dot_general(f_ref[...], wp_ref[...], (((1,), (1,)), ((), ())),
                        preferred_element_type=jnp.float32)
    x = lax.dot_general(x, w1_ref[...], (((1,), (1,)), ((), ())),
                        preferred_element_type=jnp.float32) + b1_ref[...]
    x = jnp.maximum(x, 0.0)
    m = jnp.mean(x, axis=1, keepdims=True)
    d = x - m
    s = jnp.sqrt(jnp.sum(d * d, axis=1, keepdims=True) / (x.shape[1] - 1))
    y = d / s
    o_ref[...] = jnp.where(jnp.isnan(y), 0.0, y)


def _front(feat, wp, w1, b1r, block_rows):
    n = feat.shape[0]
    return pl.pallas_call(
        _front_body,
        grid=(n // block_rows,),
        in_specs=[
            pl.BlockSpec((block_rows, 128), lambda i: (i, 0)),
            pl.BlockSpec((32, 128), lambda i: (0, 0)),
            pl.BlockSpec((32, 32), lambda i: (0, 0)),
            pl.BlockSpec((1, 32), lambda i: (0, 0)),
        ],
        out_specs=pl.BlockSpec((block_rows, 32), lambda i: (i, 0)),
        out_shape=jax.ShapeDtypeStruct((n, 32), jnp.float32),
    )(feat, wp, w1, b1r)


def _combine_body(a_ref, b_ref, c_ref, d_ref, e_ref, coe_ref, o_ref):
    o_ref[...] = (coe_ref[0] * a_ref[...] + coe_ref[1] * b_ref[...]
                  + coe_ref[2] * c_ref[...] + coe_ref[3] * d_ref[...]
                  + coe_ref[4] * e_ref[...])


_BR = 2176


def _half_specs(nblk):
    return [
        pl.BlockSpec((_BR, 32), lambda i: (i, 0)),
        pl.BlockSpec((_BR, 32), lambda i, nb=nblk: (i + nb, 0)),
    ]


def _combine(h, t1, t2, coe, n_p):
    n = h.shape[0]
    nblk = n_p // _BR
    t1a, t1b = _half_specs(nblk)
    t2a, t2b = _half_specs(nblk)
    return pl.pallas_call(
        _combine_body,
        grid=(nblk,),
        in_specs=[pl.BlockSpec((_BR, 32), lambda i: (i, 0)),
                  t1a, t1b, t2a, t2b,
                  pl.BlockSpec(memory_space=pltpu.SMEM)],
        out_specs=pl.BlockSpec((_BR, 32), lambda i: (i, 0)),
        out_shape=jax.ShapeDtypeStruct((n, 32), jnp.float32),
    )(h, t1, t1, t2, t2, coe)


def _proj_body(a_ref, b_ref, c_ref, d_ref, e_ref, coe_ref, w2_ref, b2_ref,
               o_ref):
    res = (coe_ref[0] * a_ref[...] + coe_ref[1] * b_ref[...]
           + coe_ref[2] * c_ref[...] + coe_ref[3] * d_ref[...]
           + coe_ref[4] * e_ref[...])
    o_ref[...] = lax.dot_general(res, w2_ref[...], (((1,), (1,)), ((), ())),
                                 preferred_element_type=jnp.float32) + b2_ref[...]


def _proj(h, u1, u2, coe, w2, b2r, n_p):
    n = h.shape[0]
    nblk = n_p // _BR
    u1a, u1b = _half_specs(nblk)
    u2a, u2b = _half_specs(nblk)
    return pl.pallas_call(
        _proj_body,
        grid=(nblk,),
        in_specs=[pl.BlockSpec((_BR, 32), lambda i: (i, 0)),
                  u1a, u1b, u2a, u2b,
                  pl.BlockSpec(memory_space=pltpu.SMEM),
                  pl.BlockSpec((16, 32), lambda i: (0, 0)),
                  pl.BlockSpec((1, 16), lambda i: (0, 0))],
        out_specs=pl.BlockSpec((_BR, 16), lambda i: (i, 0)),
        out_shape=jax.ShapeDtypeStruct((n, 16), jnp.float32),
    )(h, u1, u1, u2, u2, coe, w2, b2r)


def _bil_body(le_ref, re_ref, mid_ref, w0_ref, w1_ref, o_ref):
    le = le_ref[...]
    re = re_ref[...]
    p0 = jnp.sum(lax.dot_general(le, w0_ref[...], (((1,), (0,)), ((), ())),
                                 preferred_element_type=jnp.float32) * re,
                 axis=1, keepdims=True)
    p1 = jnp.sum(lax.dot_general(le, w1_ref[...], (((1,), (0,)), ((), ())),
                                 preferred_element_type=jnp.float32) * re,
                 axis=1, keepdims=True)
    o_ref[...] = jnp.where(mid_ref[...] == 0, p0, p1)


def _bilinear(le, re, mid2, w0, w1):
    b = le.shape[0]
    return pl.pallas_call(
        _bil_body,
        out_shape=jax.ShapeDtypeStruct((b, 1), jnp.float32),
    )(le, re, mid2, w0, w1)


def _scale_chunk(ed_s, buf, j):
    for g in range(8):
        v16 = plsc.bitcast(ed_s[16 + j, pl.ds(g * 16, 16)], jnp.float32)
        for i in range(16):
            e = g * 16 + i
            s = v16[i]
            buf[e, 0:16] = buf[e, 0:16] * s
            buf[e, 16:32] = buf[e, 16:32] * s


def _make_spmm_body(use_off):
    def body(h_hbm, ed_hbm, z_hbm, out_hbm,
             ed_s, b0, b1, b2, b3, acc, s0, s1, s2, s3):
        ci = lax.axis_index("c")
        ti = lax.axis_index("s")
        n = z_hbm.shape[0]
        rpt = n // N_SUB
        pltpu.sync_copy(z_hbm.at[pl.ds(ti * rpt, rpt)],
                        acc.at[pl.ds(ti * rpt, rpt)])
        plsc.subcore_barrier()

        tsup = ed_hbm.shape[0] // (N_CORES * N_SUB)
        base_sup = (ci * N_SUB + ti) * tsup
        off = ci * n
        bufs = (b0, b1, b2, b3)
        sems = (s0, s1, s2, s3)

        def sup(k, carry):
            pltpu.sync_copy(ed_hbm.at[base_sup + k], ed_s)
            if use_off:
                for r in range(8):
                    for g in range(8):
                        sl = pl.ds(g * 16, 16)
                        ed_s[8 + r, sl] = ed_s[8 + r, sl] + off
            cps = [None] * 8
            for j in range(3):
                cps[j] = pltpu.async_copy(h_hbm.at[ed_s.at[8 + j]],
                                          bufs[j], sems[j])
            for j in range(8):
                if j + 3 < 8:
                    cps[j + 3] = pltpu.async_copy(
                        h_hbm.at[ed_s.at[11 + j]], bufs[(j + 3) % 4],
                        sems[(j + 3) % 4])
                cps[j].wait()
                buf = bufs[j % 4]
                _scale_chunk(ed_s, buf, j)
                pltpu.sync_copy(buf, acc.at[ed_s.at[j]], add=True)
            return carry

        lax.fori_loop(0, tsup, sup, 0)
        plsc.subcore_barrier()
        pltpu.sync_copy(acc.at[pl.ds(ti * rpt, rpt)],
                        out_hbm.at[pl.ds(ci * n + ti * rpt, rpt)])

    return body


def _spmm2(src, ed, zeros, use_off):
    n = zeros.shape[0]
    mesh = plsc.VectorSubcoreMesh(core_axis_name="c", subcore_axis_name="s",
                                  num_cores=N_CORES, num_subcores=N_SUB)
    f = pl.kernel(
        _make_spmm_body(use_off),
        out_type=jax.ShapeDtypeStruct((2 * n, 32), jnp.float32),
        mesh=mesh,
        scratch_types=[
            pltpu.VMEM((24, 128), jnp.int32),
            pltpu.VMEM((CH, 32), jnp.float32),
            pltpu.VMEM((CH, 32), jnp.float32),
            pltpu.VMEM((CH, 32), jnp.float32),
            pltpu.VMEM((CH, 32), jnp.float32),
            pltpu.VMEM_SHARED((n, 32), jnp.float32),
            pltpu.SemaphoreType.DMA,
            pltpu.SemaphoreType.DMA,
            pltpu.SemaphoreType.DMA,
            pltpu.SemaphoreType.DMA,
        ],
        compiler_params=pltpu.CompilerParams(use_tc_tiling_on_sc=False,
                                             needs_layout_passes=False),
    )
    return f(src, ed, zeros)


def _pairs_body(l_hbm, idx_hbm, out_hbm, idxv, rows, sem):
    ci = lax.axis_index("c")
    ti = lax.axis_index("s")
    w = ti * N_CORES + ci
    pltpu.sync_copy(idx_hbm.at[pl.ds(w * 4, 4)], idxv)
    for j in range(4):
        pltpu.async_copy(l_hbm.at[idxv.at[j]], rows, sem).wait()
        pltpu.sync_copy(rows, out_hbm.at[pl.ds(w * 512 + j * 128, 128)])


def _pair_gather(logits, idx2d):
    mesh = plsc.VectorSubcoreMesh(core_axis_name="c", subcore_axis_name="s",
                                  num_cores=N_CORES, num_subcores=N_SUB)
    f = pl.kernel(
        _pairs_body,
        out_type=jax.ShapeDtypeStruct((idx2d.size, 16), jnp.float32),
        mesh=mesh,
        scratch_types=[
            pltpu.VMEM((4, 128), jnp.int32),
            pltpu.VMEM((128, 16), jnp.float32),
            pltpu.SemaphoreType.DMA,
        ],
        compiler_params=pltpu.CompilerParams(use_tc_tiling_on_sc=False,
                                             needs_layout_passes=False),
    )
    return f(logits, idx2d)


def kernel(feat_A, feat_B, ei_AA, ei_AB, ei_BA, val_AA, val_AB, val_BA,
           left, right, mid, WpA, WpB, W1, b1, W2, b2, coe, Wdec):
    n_a = feat_A.shape[0]
    n_b = feat_B.shape[0]
    n = n_a + n_b
    e = val_AA.shape[0]

    tsup = -(-e // (N_SUB * SUPE))
    ept = tsup * SUPE
    pad = N_SUB * ept - e

    def pad_i(a):
        return jnp.concatenate([a, jnp.zeros((pad,), a.dtype)]) if pad else a

    xa = _front(feat_A, WpA, W1, b1.reshape(1, -1), 1200)
    xb = _front(feat_B, WpB, W1, b1.reshape(1, -1), 1000)
    x = jnp.concatenate([xa, xb], axis=0)

    n_p = -(-n // (8 * N_SUB)) * (8 * N_SUB)
    zeros = jnp.zeros((n_p, 32), jnp.float32)

    def pack_one(ei, v):
        r = pad_i(ei[0])
        c = pad_i(ei[1])
        vb = lax.bitcast_convert_type(pad_i(v), jnp.int32)
        t = r.shape[0] // SUPE
        return jnp.concatenate(
            [r.reshape(t, 8, 128), c.reshape(t, 8, 128),
             vb.reshape(t, 8, 128)], axis=1)

    ed_a = pack_one(ei_AA, val_AA)
    ed_b = pack_one(ei_AB, val_AB)
    ed_c = pack_one(ei_BA, val_BA)
    ed_1 = jnp.concatenate([ed_a, ed_b], axis=0)
    ed_2 = jnp.concatenate([ed_a, ed_c], axis=0)

    t1 = _spmm2(x, ed_1, zeros, False)
    t2 = _spmm2(t1, ed_1, zeros, True)
    res1 = _combine(x, t1, t2, coe, n_p)

    u1 = _spmm2(res1, ed_2, zeros, False)
    u2 = _spmm2(u1, ed_2, zeros, True)
    logits = _proj(res1, u1, u2, coe, W2, b2.reshape(1, -1), n_p)

    npair = left.shape[0]
    idx2d = jnp.concatenate([left, right]).reshape(-1, 128)
    lr = _pair_gather(logits, idx2d)
    out = _bilinear(lr[:npair], lr[npair:], mid.reshape(-1, 1),
                    Wdec[0], Wdec[1])
    return out.reshape(npair)


# speedup vs baseline: 1.0087x; 1.0087x over previous
import jax
import jax.numpy as jnp
from jax import lax
from jax.experimental import pallas as pl
from jax.experimental.pallas import tpu as pltpu
from jax.experimental.pallas import tpu_sc as plsc

N_CORES = 2
N_SUB = 16
CH = 128
SUPE = 1024


def _front_body(f_ref, wp_ref, w1_ref, b1_ref, o_ref):
    x = lax.dot_general(f_ref[...], wp_ref[...], (((1,), (1,)), ((), ())),
                        preferred_element_type=jnp.float32)
    x = lax.dot_general(x, w1_ref[...], (((1,), (1,)), ((), ())),
                        preferred_element_type=jnp.float32) + b1_ref[...]
    x = jnp.maximum(x, 0.0)
    m = jnp.mean(x, axis=1, keepdims=True)
    d = x - m
    s = jnp.sqrt(jnp.sum(d * d, axis=1, keepdims=True) / (x.shape[1] - 1))
    y = d / s
    o_ref[...] = jnp.where(jnp.isnan(y), 0.0, y)


def _front(feat, wp, w1, b1r, block_rows):
    n = feat.shape[0]
    return pl.pallas_call(
        _front_body,
        grid=(n // block_rows,),
        in_specs=[
            pl.BlockSpec((block_rows, 128), lambda i: (i, 0)),
            pl.BlockSpec((32, 128), lambda i: (0, 0)),
            pl.BlockSpec((32, 32), lambda i: (0, 0)),
            pl.BlockSpec((1, 32), lambda i: (0, 0)),
        ],
        out_specs=pl.BlockSpec((block_rows, 32), lambda i: (i, 0)),
        out_shape=jax.ShapeDtypeStruct((n, 32), jnp.float32),
    )(feat, wp, w1, b1r)


def _combine_body(a_ref, b_ref, c_ref, d_ref, e_ref, coe_ref, o_ref):
    o_ref[...] = (coe_ref[0] * a_ref[...] + coe_ref[1] * b_ref[...]
                  + coe_ref[2] * c_ref[...] + coe_ref[3] * d_ref[...]
                  + coe_ref[4] * e_ref[...])


_BR = 2176


def _half_specs(nblk):
    return [
        pl.BlockSpec((_BR, 32), lambda i: (i, 0)),
        pl.BlockSpec((_BR, 32), lambda i, nb=nblk: (i + nb, 0)),
    ]


def _combine(h, t1, t2, coe, n_p):
    n = h.shape[0]
    nblk = n_p // _BR
    t1a, t1b = _half_specs(nblk)
    t2a, t2b = _half_specs(nblk)
    return pl.pallas_call(
        _combine_body,
        grid=(nblk,),
        in_specs=[pl.BlockSpec((_BR, 32), lambda i: (i, 0)),
                  t1a, t1b, t2a, t2b,
                  pl.BlockSpec(memory_space=pltpu.SMEM)],
        out_specs=pl.BlockSpec((_BR, 32), lambda i: (i, 0)),
        out_shape=jax.ShapeDtypeStruct((n, 32), jnp.float32),
    )(h, t1, t1, t2, t2, coe)


def _proj_body(a_ref, b_ref, c_ref, d_ref, e_ref, coe_ref, w2_ref, b2_ref,
               o_ref):
    res = (coe_ref[0] * a_ref[...] + coe_ref[1] * b_ref[...]
           + coe_ref[2] * c_ref[...] + coe_ref[3] * d_ref[...]
           + coe_ref[4] * e_ref[...])
    o_ref[...] = lax.dot_general(res, w2_ref[...], (((1,), (1,)), ((), ())),
                                 preferred_element_type=jnp.float32) + b2_ref[...]


def _proj(h, u1, u2, coe, w2, b2r, n_p):
    n = h.shape[0]
    nblk = n_p // _BR
    u1a, u1b = _half_specs(nblk)
    u2a, u2b = _half_specs(nblk)
    return pl.pallas_call(
        _proj_body,
        grid=(nblk,),
        in_specs=[pl.BlockSpec((_BR, 32), lambda i: (i, 0)),
                  u1a, u1b, u2a, u2b,
                  pl.BlockSpec(memory_space=pltpu.SMEM),
                  pl.BlockSpec((16, 32), lambda i: (0, 0)),
                  pl.BlockSpec((1, 16), lambda i: (0, 0))],
        out_specs=pl.BlockSpec((_BR, 16), lambda i: (i, 0)),
        out_shape=jax.ShapeDtypeStruct((n, 16), jnp.float32),
    )(h, u1, u1, u2, u2, coe, w2, b2r)


def _bil_body(le_ref, re_ref, mid_ref, w0_ref, w1_ref, o_ref):
    le = le_ref[...]
    re = re_ref[...]
    p0 = jnp.sum(lax.dot_general(le, w0_ref[...], (((1,), (0,)), ((), ())),
                                 preferred_element_type=jnp.float32) * re,
                 axis=1, keepdims=True)
    p1 = jnp.sum(lax.dot_general(le, w1_ref[...], (((1,), (0,)), ((), ())),
                                 preferred_element_type=jnp.float32) * re,
                 axis=1, keepdims=True)
    o_ref[...] = jnp.where(mid_ref[...] == 0, p0, p1)


def _bilinear(le, re, mid2, w0, w1):
    b = le.shape[0]
    return pl.pallas_call(
        _bil_body,
        out_shape=jax.ShapeDtypeStruct((b, 1), jnp.float32),
    )(le, re, mid2, w0, w1)


def _scale_chunk(ed_s, buf, j):
    for g in range(8):
        v16 = plsc.bitcast(ed_s[16 + j, pl.ds(g * 16, 16)], jnp.float32)
        for i in range(16):
            e = g * 16 + i
            s = v16[i]
            buf[e, 0:16] = buf[e, 0:16] * s
            buf[e, 16:32] = buf[e, 16:32] * s


def _make_spmm_body(use_off):
    def body(h_hbm, ed_hbm, z_hbm, out_hbm,
             ed_s, b0, b1, b2, acc, s0, s1, s2):
        ci = lax.axis_index("c")
        ti = lax.axis_index("s")
        n = z_hbm.shape[0]
        rpt = n // N_SUB
        pltpu.sync_copy(z_hbm.at[pl.ds(ti * rpt, rpt)],
                        acc.at[pl.ds(ti * rpt, rpt)])
        plsc.subcore_barrier()

        tsup = ed_hbm.shape[0] // (N_CORES * N_SUB)
        base_sup = (ci * N_SUB + ti) * tsup
        off = ci * n
        bufs = (b0, b1, b2)
        sems = (s0, s1, s2)

        def sup(k, carry):
            pltpu.sync_copy(ed_hbm.at[base_sup + k], ed_s)
            if use_off:
                for r in range(8):
                    for g in range(8):
                        sl = pl.ds(g * 16, 16)
                        ed_s[8 + r, sl] = ed_s[8 + r, sl] + off
            cps = [None] * 8
            cps[0] = pltpu.async_copy(h_hbm.at[ed_s.at[8]], b0, s0)
            cps[1] = pltpu.async_copy(h_hbm.at[ed_s.at[9]], b1, s1)
            for j in range(8):
                if j + 2 < 8:
                    cps[j + 2] = pltpu.async_copy(
                        h_hbm.at[ed_s.at[10 + j]], bufs[(j + 2) % 3],
                        sems[(j + 2) % 3])
                cps[j].wait()
                buf = bufs[j % 3]
                _scale_chunk(ed_s, buf, j)
                pltpu.sync_copy(buf, acc.at[ed_s.at[j]], add=True)
            return carry

        lax.fori_loop(0, tsup, sup, 0)
        plsc.subcore_barrier()
        pltpu.sync_copy(acc.at[pl.ds(ti * rpt, rpt)],
                        out_hbm.at[pl.ds(ci * n + ti * rpt, rpt)])

    return body


def _spmm2(src, ed, zeros, use_off):
    n = zeros.shape[0]
    mesh = plsc.VectorSubcoreMesh(core_axis_name="c", subcore_axis_name="s",
                                  num_cores=N_CORES, num_subcores=N_SUB)
    f = pl.kernel(
        _make_spmm_body(use_off),
        out_type=jax.ShapeDtypeStruct((2 * n, 32), jnp.float32),
        mesh=mesh,
        scratch_types=[
            pltpu.VMEM((24, 128), jnp.int32),
            pltpu.VMEM((CH, 32), jnp.float32),
            pltpu.VMEM((CH, 32), jnp.float32),
            pltpu.VMEM((CH, 32), jnp.float32),
            pltpu.VMEM_SHARED((n, 32), jnp.float32),
            pltpu.SemaphoreType.DMA,
            pltpu.SemaphoreType.DMA,
            pltpu.SemaphoreType.DMA,
        ],
        compiler_params=pltpu.CompilerParams(use_tc_tiling_on_sc=False,
                                             needs_layout_passes=False),
    )
    return f(src, ed, zeros)


def _pairs_body(l_hbm, idx_hbm, out_hbm, idxv, rows, sem):
    ci = lax.axis_index("c")
    ti = lax.axis_index("s")
    w = ti * N_CORES + ci
    pltpu.sync_copy(idx_hbm.at[pl.ds(w * 4, 4)], idxv)
    for j in range(4):
        pltpu.async_copy(l_hbm.at[idxv.at[j]], rows, sem).wait()
        pltpu.sync_copy(rows, out_hbm.at[pl.ds(w * 512 + j * 128, 128)])


def _pair_gather(logits, idx2d):
    mesh = plsc.VectorSubcoreMesh(core_axis_name="c", subcore_axis_name="s",
                                  num_cores=N_CORES, num_subcores=N_SUB)
    f = pl.kernel(
        _pairs_body,
        out_type=jax.ShapeDtypeStruct((idx2d.size, 16), jnp.float32),
        mesh=mesh,
        scratch_types=[
            pltpu.VMEM((4, 128), jnp.int32),
            pltpu.VMEM((128, 16), jnp.float32),
            pltpu.SemaphoreType.DMA,
        ],
        compiler_params=pltpu.CompilerParams(use_tc_tiling_on_sc=False,
                                             needs_layout_passes=False),
    )
    return f(logits, idx2d)


def kernel(feat_A, feat_B, ei_AA, ei_AB, ei_BA, val_AA, val_AB, val_BA,
           left, right, mid, WpA, WpB, W1, b1, W2, b2, coe, Wdec):
    n_a = feat_A.shape[0]
    n_b = feat_B.shape[0]
    n = n_a + n_b
    e = val_AA.shape[0]

    tsup = -(-e // (N_SUB * SUPE))
    ept = tsup * SUPE
    pad = N_SUB * ept - e

    def pad_i(a):
        return jnp.concatenate([a, jnp.zeros((pad,), a.dtype)]) if pad else a

    xa = _front(feat_A, WpA, W1, b1.reshape(1, -1), 1200)
    xb = _front(feat_B, WpB, W1, b1.reshape(1, -1), 1000)
    x = jnp.concatenate([xa, xb], axis=0)

    n_p = -(-n // (8 * N_SUB)) * (8 * N_SUB)
    zeros = jnp.zeros((n_p, 32), jnp.float32)

    def pack_one(ei, v):
        r = pad_i(ei[0])
        c = pad_i(ei[1])
        vb = lax.bitcast_convert_type(pad_i(v), jnp.int32)
        t = r.shape[0] // SUPE
        return jnp.concatenate(
            [r.reshape(t, 8, 128), c.reshape(t, 8, 128),
             vb.reshape(t, 8, 128)], axis=1)

    ed_a = pack_one(ei_AA, val_AA)
    ed_b = pack_one(ei_AB, val_AB)
    ed_c = pack_one(ei_BA, val_BA)
    ed_1 = jnp.concatenate([ed_a, ed_b], axis=0)
    ed_2 = jnp.concatenate([ed_a, ed_c], axis=0)

    t1 = _spmm2(x, ed_1, zeros, False)
    t2 = _spmm2(t1, ed_1, zeros, True)
    res1 = _combine(x, t1, t2, coe, n_p)

    u1 = _spmm2(res1, ed_2, zeros, False)
    u2 = _spmm2(u1, ed_2, zeros, True)
    logits = _proj(res1, u1, u2, coe, W2, b2.reshape(1, -1), n_p)

    npair = left.shape[0]
    idx2d = jnp.concatenate([left, right]).reshape(-1, 128)
    lr = _pair_gather(logits, idx2d)
    out = _bilinear(lr[:npair], lr[npair:], mid.reshape(-1, 1),
                    Wdec[0], Wdec[1])
    return out.reshape(npair)
